# baseline (speedup 1.0000x reference)
import functools

import jax
import jax.numpy as jnp
from jax import lax
from jax.experimental import pallas as pl
from jax.experimental.pallas import tpu as pltpu
from jax.experimental.pallas import tpu_sc as plsc

_CUT2 = 25.0
_K = 32
_N = 4096
_BR = 128
_W = 128
_NH = _N // 2


def _topk_body(off, posb_ref, posbt_ref, fillv_ref, filli_ref,
               lo_ref, hi_ref, src_ref, ew_ref):
    b = pl.program_id(0)
    px = posbt_ref[0:1, :]
    py = posbt_ref[1:2, :]
    pz = posbt_ref[2:3, :]
    brow = posbt_ref[3:4, :]
    rows = (off + b) * _BR + lax.broadcasted_iota(jnp.int32, (1, _BR), 1)
    kiota = lax.broadcasted_iota(jnp.int32, (_K, 1), 0)
    tail0v = jnp.broadcast_to(fillv_ref[0], (_K, _BR))
    tail0i = jnp.broadcast_to(filli_ref[0], (_K, _BR))
    lo = lo_ref[b]
    hi = hi_ref[b]

    def chunk(c, carry):
        tailv, taili = carry
        base = pl.multiple_of(c * _W, _W)
        xcol = posb_ref[pl.ds(base, _W), 0:1]
        ycol = posb_ref[pl.ds(base, _W), 1:2]
        zcol = posb_ref[pl.ds(base, _W), 2:3]
        bcol = posb_ref[pl.ds(base, _W), 3:4]
        dx = xcol - px
        dy = ycol - py
        dz = zcol - pz
        d2 = dx * dx + dy * dy + dz * dz
        colsw = base + lax.broadcasted_iota(jnp.int32, (_W, 1), 0)
        ok = (bcol == brow) & (colsw != rows) & (d2 <= _CUT2)
        vc = jnp.where(ok, d2, 100.0 + colsw.astype(jnp.float32))

        pvv, piv = tailv, taili
        for t in range(_K):
            m = jnp.minimum(jnp.min(vc, axis=0, keepdims=True),
                            jnp.min(tailv, axis=0, keepdims=True))
            isc = vc == m
            ist = tailv == m
            pick = jnp.minimum(
                jnp.min(jnp.where(isc, colsw, _N), axis=0, keepdims=True),
                jnp.min(jnp.where(ist, taili, _N), axis=0, keepdims=True))
            sel = kiota == t
            pvv = jnp.where(sel, m, pvv)
            piv = jnp.where(sel, pick, piv)
            vc = jnp.where(colsw == pick, jnp.inf, vc)
            tailv = jnp.where(taili == pick, jnp.inf, tailv)
        return pvv, piv

    tailv, taili = lax.fori_loop(lo, hi, chunk, (tail0v, tail0i))
    src_ref[...] = taili
    ew_ref[...] = jnp.where(tailv < 100.0, jnp.sqrt(tailv), 0.0)


def _topk_half(posb, posbT, fill_val, fill_idx, lo_chunk, hi_chunk, half):
    nb = _NH // _BR
    off = half * nb
    body = functools.partial(_topk_body, off)
    return pl.pallas_call(
        body,
        grid=(nb,),
        in_specs=[
            pl.BlockSpec((_N, 4), lambda b: (0, 0)),
            pl.BlockSpec((4, _BR), lambda b, o=off: (0, b + o)),
            pl.BlockSpec((1, _K, 1), lambda b: (b, 0, 0)),
            pl.BlockSpec((1, _K, 1), lambda b: (b, 0, 0)),
            pl.BlockSpec(memory_space=pltpu.SMEM),
            pl.BlockSpec(memory_space=pltpu.SMEM),
        ],
        out_specs=[
            pl.BlockSpec((_K, _BR), lambda b: (0, b)),
            pl.BlockSpec((_K, _BR), lambda b: (0, b)),
        ],
        out_shape=[
            jax.ShapeDtypeStruct((_K, _NH), jnp.int32),
            jax.ShapeDtypeStruct((_K, _NH), jnp.float32),
        ],
    )(posb, posbT,
      fill_val[off:off + nb].reshape(nb, _K, 1),
      fill_idx[off:off + nb].reshape(nb, _K, 1),
      lo_chunk[off:off + nb], hi_chunk[off:off + nb])


def _block_ranges(batch):
    nb = _N // _BR
    starts = jnp.searchsorted(batch, jnp.arange(8, dtype=jnp.int32),
                              side="left").astype(jnp.int32)
    ends = jnp.searchsorted(batch, jnp.arange(8, dtype=jnp.int32),
                            side="right").astype(jnp.int32)
    g_first = batch[0::_BR]
    g_last = batch[_BR - 1::_BR]
    blk_lo = starts[g_first]
    blk_hi = ends[g_last]
    lo_chunk = blk_lo // _W
    hi_chunk = (blk_hi + _W - 1) // _W
    t = jnp.arange(_K, dtype=jnp.int32)[None, :]
    lo2 = blk_lo[:, None]
    hi2 = blk_hi[:, None]
    fcol = jnp.where(t < lo2, t, hi2 + (t - lo2))
    exists = fcol < _N
    fill_val = jnp.where(exists, 100.0 + fcol.astype(jnp.float32), jnp.inf)
    fill_idx = jnp.where(exists, fcol, _N - 1).astype(jnp.int32)
    return fill_val, fill_idx, lo_chunk, hi_chunk


_NW = 32
_LANES = 16
_LN = 128


def _edge_gather(posx, posy, posz, src3, dst3, ew3):
    rows = src3.shape[1]
    mesh = plsc.VectorSubcoreMesh(core_axis_name="c", subcore_axis_name="s")

    def body(posx_h, posy_h, posz_h, src_h, dst_h, ew_h,
             ox_h, oy_h, oz_h,
             px, py, pz,
             sbuf, dbuf, wbuf, gsx, gsy, gsz, gdx, gdy, gdz,
             ox, oy, oz, sem):
        wid = lax.axis_index("s") * 2 + lax.axis_index("c")
        @pl.when(lax.axis_index("s") == 0)
        def _stage():
            pltpu.sync_copy(posx_h, px)
            pltpu.sync_copy(posy_h, py)
            pltpu.sync_copy(posz_h, pz)

        plsc.subcore_barrier()
        pltpu.sync_copy(src_h.at[wid], sbuf)
        pltpu.sync_copy(dst_h.at[wid], dbuf)
        pltpu.sync_copy(ew_h.at[wid], wbuf)

        def fire(j, _):
            pltpu.async_copy(px.at[sbuf.at[j]], gsx.at[j], sem)
            pltpu.async_copy(py.at[sbuf.at[j]], gsy.at[j], sem)
            pltpu.async_copy(pz.at[sbuf.at[j]], gsz.at[j], sem)
            pltpu.async_copy(px.at[dbuf.at[j]], gdx.at[j], sem)
            pltpu.async_copy(py.at[dbuf.at[j]], gdy.at[j], sem)
            pltpu.async_copy(pz.at[dbuf.at[j]], gdz.at[j], sem)
            return 0

        lax.fori_loop(0, rows, fire, 0)
        for buf in (gsx, gsy, gsz, gdx, gdy, gdz):
            pltpu.make_async_copy(ox_h.at[wid], buf, sem).wait()

        def compute(j, _):
            for t in range(_LN // _LANES):
                sl = pl.ds(t * _LANES, _LANES)
                v = wbuf[j, sl] > 0.0
                ox[j, sl] = jnp.where(v, gsx[j, sl] - gdx[j, sl], 0.0)
                oy[j, sl] = jnp.where(v, gsy[j, sl] - gdy[j, sl], 0.0)
                oz[j, sl] = jnp.where(v, gsz[j, sl] - gdz[j, sl], 0.0)
            return 0

        lax.fori_loop(0, rows, compute, 0)
        pltpu.sync_copy(ox, ox_h.at[wid])
        pltpu.sync_copy(oy, oy_h.at[wid])
        pltpu.sync_copy(oz, oz_h.at[wid])

    f = functools.partial(
        pl.kernel,
        mesh=mesh,
        out_type=[jax.ShapeDtypeStruct((_NW, rows, _LN), jnp.float32)] * 3,
        scratch_types=[
            pltpu.VMEM_SHARED((_N,), jnp.float32),
            pltpu.VMEM_SHARED((_N,), jnp.float32),
            pltpu.VMEM_SHARED((_N,), jnp.float32),
            pltpu.VMEM((rows, _LN), jnp.int32),
            pltpu.VMEM((rows, _LN), jnp.int32),
        ] + [pltpu.VMEM((rows, _LN), jnp.float32)] * 10 + [
            pltpu.SemaphoreType.DMA,
        ],
    )(body)
    return f(posx, posy, posz, src3, dst3, ew3)


def kernel(pos, batch):
    bf = batch.astype(jnp.float32).reshape(_N, 1)
    posb = jnp.concatenate([pos, bf], axis=1)
    posbT = posb.T
    ranges = _block_ranges(batch)

    posx, posy, posz = pos[:, 0], pos[:, 1], pos[:, 2]
    eh = _NH * _K
    rows_h = eh // (_NW * _LN)
    dst = lax.broadcasted_iota(jnp.int32, (_N, _K), 0).reshape(-1)

    srcs, ews, evs = [], [], []
    for half in (0, 1):
        srcT, ewT = _topk_half(posb, posbT, *ranges, half)
        src_flat = srcT.T.reshape(-1)
        ew_flat = ewT.T.reshape(-1)
        dst_h = dst[half * eh:(half + 1) * eh]
        evx, evy, evz = _edge_gather(
            posx, posy, posz,
            src_flat.reshape(_NW, rows_h, _LN),
            dst_h.reshape(_NW, rows_h, _LN),
            ew_flat.reshape(_NW, rows_h, _LN))
        srcs.append(src_flat)
        ews.append(ew_flat)
        evs.append(jnp.stack([evx.reshape(-1), evy.reshape(-1),
                              evz.reshape(-1)], axis=1))

    src_flat = jnp.concatenate(srcs)
    ew_flat = jnp.concatenate(ews)
    ev = jnp.concatenate(evs, axis=0)
    n = _N
    loop_idx = jnp.arange(n, dtype=jnp.int32)
    edge_index = jnp.concatenate(
        [jnp.stack([src_flat, dst]), jnp.stack([loop_idx, loop_idx])], axis=1)
    ew_full = jnp.concatenate([ew_flat, jnp.zeros((n,), jnp.float32)])
    ev_full = jnp.concatenate([ev, jnp.zeros((n, 3), jnp.float32)], axis=0)
    return edge_index, ew_full, ev_full


# speedup vs baseline: 1.1400x; 1.1400x over previous
import functools

import jax
import jax.numpy as jnp
from jax import lax
from jax.experimental import pallas as pl
from jax.experimental.pallas import tpu as pltpu
from jax.experimental.pallas import tpu_sc as plsc

_CUT2 = 25.0
_K = 32
_N = 4096
_BR = 128
_W = 256
_NH = _N // 2


def _topk_body(off, posb_ref, posbt_ref, fillv_ref, filli_ref,
               lo_ref, hi_ref, src_ref, ew_ref):
    b = pl.program_id(0)
    px = posbt_ref[0:1, :]
    py = posbt_ref[1:2, :]
    pz = posbt_ref[2:3, :]
    brow = posbt_ref[3:4, :]
    rows = (off + b) * _BR + lax.broadcasted_iota(jnp.int32, (1, _BR), 1)
    kiota = lax.broadcasted_iota(jnp.int32, (_K, 1), 0)
    tail0v = jnp.broadcast_to(fillv_ref[0], (_K, _BR))
    tail0i = jnp.broadcast_to(filli_ref[0], (_K, _BR))
    lo = lo_ref[b]
    hi = hi_ref[b]

    def chunk(c, carry):
        tailv, taili = carry
        base = pl.multiple_of(c * _W, _W)
        xcol = posb_ref[pl.ds(base, _W), 0:1]
        ycol = posb_ref[pl.ds(base, _W), 1:2]
        zcol = posb_ref[pl.ds(base, _W), 2:3]
        bcol = posb_ref[pl.ds(base, _W), 3:4]
        dx = xcol - px
        dy = ycol - py
        dz = zcol - pz
        d2 = dx * dx + dy * dy + dz * dz
        colsw = base + lax.broadcasted_iota(jnp.int32, (_W, 1), 0)
        ok = (bcol == brow) & (colsw != rows) & (d2 <= _CUT2)
        vc = jnp.where(ok, d2, 100.0 + colsw.astype(jnp.float32))

        pvv, piv = tailv, taili
        for t in range(_K):
            m = jnp.minimum(jnp.min(vc, axis=0, keepdims=True),
                            jnp.min(tailv, axis=0, keepdims=True))
            isc = vc == m
            ist = tailv == m
            pick = jnp.minimum(
                jnp.min(jnp.where(isc, colsw, _N), axis=0, keepdims=True),
                jnp.min(jnp.where(ist, taili, _N), axis=0, keepdims=True))
            sel = kiota == t
            pvv = jnp.where(sel, m, pvv)
            piv = jnp.where(sel, pick, piv)
            vc = jnp.where(colsw == pick, jnp.inf, vc)
            tailv = jnp.where(taili == pick, jnp.inf, tailv)
        return pvv, piv

    tailv, taili = lax.fori_loop(lo, hi, chunk, (tail0v, tail0i))
    src_ref[...] = taili
    ew_ref[...] = jnp.where(tailv < 100.0, jnp.sqrt(tailv), 0.0)


def _topk_half(posb, posbT, fill_val, fill_idx, lo_chunk, hi_chunk, half):
    nb = _NH // _BR
    off = half * nb
    body = functools.partial(_topk_body, off)
    return pl.pallas_call(
        body,
        grid=(nb,),
        in_specs=[
            pl.BlockSpec((_N, 4), lambda b: (0, 0)),
            pl.BlockSpec((4, _BR), lambda b, o=off: (0, b + o)),
            pl.BlockSpec((1, _K, 1), lambda b: (b, 0, 0)),
            pl.BlockSpec((1, _K, 1), lambda b: (b, 0, 0)),
            pl.BlockSpec(memory_space=pltpu.SMEM),
            pl.BlockSpec(memory_space=pltpu.SMEM),
        ],
        out_specs=[
            pl.BlockSpec((_K, _BR), lambda b: (0, b)),
            pl.BlockSpec((_K, _BR), lambda b: (0, b)),
        ],
        out_shape=[
            jax.ShapeDtypeStruct((_K, _NH), jnp.int32),
            jax.ShapeDtypeStruct((_K, _NH), jnp.float32),
        ],
    )(posb, posbT,
      fill_val[off:off + nb].reshape(nb, _K, 1),
      fill_idx[off:off + nb].reshape(nb, _K, 1),
      lo_chunk[off:off + nb], hi_chunk[off:off + nb])


def _block_ranges(batch):
    nb = _N // _BR
    starts = jnp.searchsorted(batch, jnp.arange(8, dtype=jnp.int32),
                              side="left").astype(jnp.int32)
    ends = jnp.searchsorted(batch, jnp.arange(8, dtype=jnp.int32),
                            side="right").astype(jnp.int32)
    g_first = batch[0::_BR]
    g_last = batch[_BR - 1::_BR]
    blk_lo = starts[g_first]
    blk_hi = ends[g_last]
    lo_chunk = blk_lo // _W
    hi_chunk = (blk_hi + _W - 1) // _W
    t = jnp.arange(_K, dtype=jnp.int32)[None, :]
    lo2 = blk_lo[:, None]
    hi2 = blk_hi[:, None]
    fcol = jnp.where(t < lo2, t, hi2 + (t - lo2))
    exists = fcol < _N
    fill_val = jnp.where(exists, 100.0 + fcol.astype(jnp.float32), jnp.inf)
    fill_idx = jnp.where(exists, fcol, _N - 1).astype(jnp.int32)
    return fill_val, fill_idx, lo_chunk, hi_chunk


_NW = 32
_LANES = 16
_LN = 128


def _edge_gather(posx, posy, posz, src3, dst3, ew3):
    rows = src3.shape[1]
    mesh = plsc.VectorSubcoreMesh(core_axis_name="c", subcore_axis_name="s")

    def body(posx_h, posy_h, posz_h, src_h, dst_h, ew_h,
             ox_h, oy_h, oz_h,
             px, py, pz,
             sbuf, dbuf, wbuf, gsx, gsy, gsz, gdx, gdy, gdz,
             ox, oy, oz, sem):
        wid = lax.axis_index("s") * 2 + lax.axis_index("c")
        @pl.when(lax.axis_index("s") == 0)
        def _stage():
            pltpu.sync_copy(posx_h, px)
            pltpu.sync_copy(posy_h, py)
            pltpu.sync_copy(posz_h, pz)

        plsc.subcore_barrier()
        pltpu.sync_copy(src_h.at[wid], sbuf)
        pltpu.sync_copy(dst_h.at[wid], dbuf)
        pltpu.sync_copy(ew_h.at[wid], wbuf)

        def fire(j, _):
            pltpu.async_copy(px.at[sbuf.at[j]], gsx.at[j], sem)
            pltpu.async_copy(py.at[sbuf.at[j]], gsy.at[j], sem)
            pltpu.async_copy(pz.at[sbuf.at[j]], gsz.at[j], sem)
            pltpu.async_copy(px.at[dbuf.at[j]], gdx.at[j], sem)
            pltpu.async_copy(py.at[dbuf.at[j]], gdy.at[j], sem)
            pltpu.async_copy(pz.at[dbuf.at[j]], gdz.at[j], sem)
            return 0

        lax.fori_loop(0, rows, fire, 0)
        for buf in (gsx, gsy, gsz, gdx, gdy, gdz):
            pltpu.make_async_copy(ox_h.at[wid], buf, sem).wait()

        def compute(j, _):
            for t in range(_LN // _LANES):
                sl = pl.ds(t * _LANES, _LANES)
                v = wbuf[j, sl] > 0.0
                ox[j, sl] = jnp.where(v, gsx[j, sl] - gdx[j, sl], 0.0)
                oy[j, sl] = jnp.where(v, gsy[j, sl] - gdy[j, sl], 0.0)
                oz[j, sl] = jnp.where(v, gsz[j, sl] - gdz[j, sl], 0.0)
            return 0

        lax.fori_loop(0, rows, compute, 0)
        pltpu.sync_copy(ox, ox_h.at[wid])
        pltpu.sync_copy(oy, oy_h.at[wid])
        pltpu.sync_copy(oz, oz_h.at[wid])

    f = functools.partial(
        pl.kernel,
        mesh=mesh,
        out_type=[jax.ShapeDtypeStruct((_NW, rows, _LN), jnp.float32)] * 3,
        scratch_types=[
            pltpu.VMEM_SHARED((_N,), jnp.float32),
            pltpu.VMEM_SHARED((_N,), jnp.float32),
            pltpu.VMEM_SHARED((_N,), jnp.float32),
            pltpu.VMEM((rows, _LN), jnp.int32),
            pltpu.VMEM((rows, _LN), jnp.int32),
        ] + [pltpu.VMEM((rows, _LN), jnp.float32)] * 10 + [
            pltpu.SemaphoreType.DMA,
        ],
    )(body)
    return f(posx, posy, posz, src3, dst3, ew3)


def kernel(pos, batch):
    bf = batch.astype(jnp.float32).reshape(_N, 1)
    posb = jnp.concatenate([pos, bf], axis=1)
    posbT = posb.T
    ranges = _block_ranges(batch)

    posx, posy, posz = pos[:, 0], pos[:, 1], pos[:, 2]
    eh = _NH * _K
    rows_h = eh // (_NW * _LN)
    dst = lax.broadcasted_iota(jnp.int32, (_N, _K), 0).reshape(-1)

    srcs, ews, evs = [], [], []
    for half in (0, 1):
        srcT, ewT = _topk_half(posb, posbT, *ranges, half)
        src_flat = srcT.T.reshape(-1)
        ew_flat = ewT.T.reshape(-1)
        dst_h = dst[half * eh:(half + 1) * eh]
        evx, evy, evz = _edge_gather(
            posx, posy, posz,
            src_flat.reshape(_NW, rows_h, _LN),
            dst_h.reshape(_NW, rows_h, _LN),
            ew_flat.reshape(_NW, rows_h, _LN))
        srcs.append(src_flat)
        ews.append(ew_flat)
        evs.append(jnp.stack([evx.reshape(-1), evy.reshape(-1),
                              evz.reshape(-1)], axis=1))

    src_flat = jnp.concatenate(srcs)
    ew_flat = jnp.concatenate(ews)
    ev = jnp.concatenate(evs, axis=0)
    n = _N
    loop_idx = jnp.arange(n, dtype=jnp.int32)
    edge_index = jnp.concatenate(
        [jnp.stack([src_flat, dst]), jnp.stack([loop_idx, loop_idx])], axis=1)
    ew_full = jnp.concatenate([ew_flat, jnp.zeros((n,), jnp.float32)])
    ev_full = jnp.concatenate([ev, jnp.zeros((n, 3), jnp.float32)], axis=0)
    return edge_index, ew_full, ev_full
